# baseline (speedup 1.0000x reference)
import functools
import numpy as np
import jax
from jax import lax
import jax.numpy as jnp
from jax.experimental import pallas as pl
from jax.experimental.pallas import tpu as pltpu
from jax.experimental.pallas import tpu_sc as plsc

_NR = 5000
_NRP = 5120
_NL = 20000
_NLP = 20096
_K = 8
_T = 10
_RS = 128
_RL = 200
_LOG2PI = float(np.log(2.0 * np.pi))
_W_OCC = 0.2
_W_MDN = 1.0
_W_INT = 0.1
_PAD_S = 65535.0

_NG = 65536
_PER_W = _NG // 32
_CHUNKS = _PER_W // 128


def _select_body(raug_ref, lmat_ref, idx_ref):
    lm = lmat_ref[...]
    l0 = lm[0:1, :]
    l1 = lm[1:2, :]
    l2c = lm[2:3, :]
    lpad = lm[4:5, :]
    lsq = l0 * l0 + l1 * l1 + l2c * l2c + lpad

    rdl = jnp.dot(raug_ref[...], lm, preferred_element_type=jnp.float32)
    s = lsq + rdl
    packed = s.astype(jnp.int32) * 32768 + jax.lax.broadcasted_iota(
        jnp.int32, s.shape, 1)

    for t in range(_T):
        m = jnp.min(packed, axis=1, keepdims=True)
        idx_ref[:, t:t + 1] = m & 32767
        packed = jnp.where(packed == m, jnp.int32(2147483647), packed)


def _select(raug, lmat):
    return pl.pallas_call(
        _select_body,
        grid=(_NRP // _RS,),
        in_specs=[
            pl.BlockSpec((_RS, 8), lambda i: (i, 0)),
            pl.BlockSpec((8, _NLP), lambda i: (0, 0)),
        ],
        out_specs=pl.BlockSpec((_RS, 16), lambda i: (i, 0)),
        out_shape=jax.ShapeDtypeStruct((_NRP, 16), jnp.int32),
    )(raug, lmat)


def _sc_gather_body(table_hbm, idx_hbm, out_hbm, idx_v, rows_a, rows_b,
                    sem_a, sem_b):
    wid = lax.axis_index("s") * 2 + lax.axis_index("c")
    pltpu.sync_copy(idx_hbm.at[pl.ds(wid * _CHUNKS, _CHUNKS)], idx_v)
    bufs = (rows_a, rows_b)
    sems = (sem_a, sem_b)
    copies = [None] * _CHUNKS
    for j in range(_CHUNKS):
        copies[j] = pltpu.async_copy(table_hbm.at[idx_v.at[j]],
                                     bufs[j % 2], sems[j % 2])
        if j > 0:
            copies[j - 1].wait()
            pltpu.sync_copy(bufs[(j - 1) % 2],
                            out_hbm.at[pl.ds(wid * _PER_W + (j - 1) * 128, 128)])
    copies[_CHUNKS - 1].wait()
    pltpu.sync_copy(bufs[(_CHUNKS - 1) % 2],
                    out_hbm.at[pl.ds(wid * _PER_W + (_CHUNKS - 1) * 128, 128)])


@functools.cache
def _sc_gather():
    return pl.kernel(
        _sc_gather_body,
        mesh=plsc.VectorSubcoreMesh(core_axis_name="c", subcore_axis_name="s"),
        out_type=jax.ShapeDtypeStruct((_NG, 128), jnp.float32),
        scratch_types=[
            pltpu.VMEM((_CHUNKS, 128), jnp.int32),
            pltpu.VMEM((128, 128), jnp.float32),
            pltpu.VMEM((128, 128), jnp.float32),
            pltpu.SemaphoreType.DMA,
            pltpu.SemaphoreType.DMA,
        ],
    )


def _gather_rows(table, idx2d):
    return _sc_gather()(table, idx2d)


def _loss_body(par_ref, gath_ref, out_ref):
    i = pl.program_id(0)

    @pl.when(i == 0)
    def _init():
        out_ref[...] = jnp.zeros((1, 1), jnp.float32)

    par = par_ref[...]
    mu = (par[:, 0:8], par[:, 8:16], par[:, 16:24])
    ls = (par[:, 24:32], par[:, 32:40], par[:, 40:48])
    mix = par[:, 48:56]
    occ = par[:, 56:64]
    mui = par[:, 64:72]
    rp = (par[:, 72:73], par[:, 73:74], par[:, 74:75])

    inv_s2 = tuple(1.0 / (jnp.exp(2.0 * l) + 1e-12) for l in ls)
    mmax = jnp.max(mix, axis=1, keepdims=True)
    lpi = mix - mmax - jnp.log(jnp.sum(jnp.exp(mix - mmax), axis=1,
                                       keepdims=True))

    gath = gath_ref[...]
    mdn_part = jnp.float32(0.0)
    int_part = jnp.float32(0.0)
    for t in range(_T):
        b = 128 * t
        g3 = gath[:, b + 3:b + 4]
        g7 = gath[:, b + 7:b + 8]
        g11 = gath[:, b + 11:b + 12]
        g15 = gath[:, b + 15:b + 16]
        gt_int = (g3 + g7 + g11 + g15) * 0.25
        y = (gath[:, b + 18:b + 19] - rp[2],
             gath[:, b + 17:b + 18] - rp[1],
             gath[:, b + 16:b + 17] - rp[0])
        quad = sum(((y[d] - mu[d]) ** 2) * inv_s2[d] + 2.0 * ls[d]
                   for d in range(3))
        logn = -0.5 * (quad + 3.0 * _LOG2PI)
        lmix = logn + lpi
        mx = jnp.max(lmix, axis=1, keepdims=True)
        e = jnp.exp(lmix - mx)
        se = jnp.sum(e, axis=1, keepdims=True)
        mdn_part += jnp.sum(mx + jnp.log(se))
        int_part += jnp.sum((e / se) * jnp.abs(mui - gt_int))

    occ_any = jnp.max(occ, axis=1)
    z = -occ_any
    occ_part = jnp.sum(jnp.maximum(z, 0.0) + jnp.log(1.0 + jnp.exp(-jnp.abs(z))))

    total = ((_W_OCC / _NR) * occ_part
             + (-_W_MDN / (_NR * _T)) * mdn_part
             + (_W_INT / (_NR * _T * _K)) * int_part)
    out_ref[...] += jnp.reshape(total, (1, 1))


def _loss(par, gath):
    return pl.pallas_call(
        _loss_body,
        grid=(_NR // _RL,),
        in_specs=[
            pl.BlockSpec((_RL, 80), lambda i: (i, 0)),
            pl.BlockSpec((_RL, 128 * _T), lambda i: (i, 0)),
        ],
        out_specs=pl.BlockSpec((1, 1), lambda i: (0, 0)),
        out_shape=jax.ShapeDtypeStruct((1, 1), jnp.float32),
    )(par, gath)


def kernel(mu_off, log_sig_off, mu_int, occ_logit, mix_logit,
           radar_indices, radar_features, lidar_indices, lidar_features):
    rpos = radar_indices[:, 1:].astype(jnp.float32)
    lpos = lidar_indices[:, 1:].astype(jnp.float32)
    padc = _NLP - _NL

    raug = jnp.concatenate(
        [-2.0 * rpos, jnp.zeros((_NR, 5), jnp.float32)], axis=1)
    raug = jnp.pad(raug, ((0, _NRP - _NR), (0, 0)))

    lmat = jnp.concatenate([
        jnp.pad(lpos.T, ((0, 0), (0, padc))),
        jnp.zeros((1, _NLP), jnp.float32),
        jnp.pad(jnp.zeros((1, _NL), jnp.float32), ((0, 0), (0, padc)),
                constant_values=_PAD_S),
        jnp.zeros((3, _NLP), jnp.float32),
    ], axis=0)

    nn_idx = _select(raug, lmat)

    table = jnp.concatenate([
        lidar_features, lpos, jnp.zeros((_NL, 109), jnp.float32),
    ], axis=1)
    idx_flat = nn_idx[:_NR, :_T].reshape(_NR * _T)
    idx2d = jnp.pad(idx_flat, (0, _NG - _NR * _T)).reshape(_NG // 128, 128)

    gath = _gather_rows(table, idx2d)
    gath = gath[:_NR * _T].reshape(_NR, 128 * _T)

    mu_t = mu_off.transpose(0, 2, 1).reshape(_NR, 24)
    ls_t = log_sig_off.transpose(0, 2, 1).reshape(_NR, 24)
    par = jnp.concatenate([
        mu_t, ls_t,
        mix_logit[..., 0], occ_logit[..., 0], mu_int[..., 0],
        rpos, jnp.zeros((_NR, 5), jnp.float32),
    ], axis=1)

    out = _loss(par, gath)
    return out[0, 0]


# speedup vs baseline: 1.4248x; 1.4248x over previous
import functools
import numpy as np
import jax
from jax import lax
import jax.numpy as jnp
from jax.experimental import pallas as pl
from jax.experimental.pallas import tpu as pltpu
from jax.experimental.pallas import tpu_sc as plsc

_NR = 5000
_NRP = 5120
_NL = 20000
_NLP = 20096
_K = 8
_T = 10
_RS = 128
_RL = 200
_LOG2PI = float(np.log(2.0 * np.pi))
_W_OCC = 0.2
_W_MDN = 1.0
_W_INT = 0.1
_PAD_S = 65535.0

_NG = 65536
_PER_W = _NG // 32
_CHUNKS = _PER_W // 128


def _select_body(raug_ref, lmat_ref, idx_ref):
    lm = lmat_ref[...]
    l0 = lm[0:1, :]
    l1 = lm[1:2, :]
    l2c = lm[2:3, :]
    lpad = lm[4:5, :]
    lsq = l0 * l0 + l1 * l1 + l2c * l2c + lpad

    rdl = jnp.dot(raug_ref[...], lm, preferred_element_type=jnp.float32)
    s = lsq + rdl
    packed = s.astype(jnp.int32) * 32768 + jax.lax.broadcasted_iota(
        jnp.int32, s.shape, 1)

    for t in range(_T):
        m = jnp.min(packed, axis=1, keepdims=True)
        idx_ref[:, t:t + 1] = m & 32767
        packed = jnp.where(packed == m, jnp.int32(2147483647), packed)


def _select(raug, lmat):
    return pl.pallas_call(
        _select_body,
        grid=(_NRP // _RS,),
        in_specs=[
            pl.BlockSpec((_RS, 8), lambda i: (i, 0)),
            pl.BlockSpec((8, _NLP), lambda i: (0, 0)),
        ],
        out_specs=pl.BlockSpec((_RS, 16), lambda i: (i, 0)),
        out_shape=jax.ShapeDtypeStruct((_NRP, 16), jnp.int32),
    )(raug, lmat)


def _sc_gather_body(table_hbm, idx_hbm, out_hbm, idx_v, rows_a, rows_b,
                    sem_a, sem_b):
    wid = lax.axis_index("s") * 2 + lax.axis_index("c")
    pltpu.sync_copy(idx_hbm.at[pl.ds(wid * _CHUNKS, _CHUNKS)], idx_v)
    bufs = (rows_a, rows_b)
    sems = (sem_a, sem_b)
    copies = [None] * _CHUNKS
    for j in range(_CHUNKS):
        copies[j] = pltpu.async_copy(table_hbm.at[idx_v.at[j]],
                                     bufs[j % 2], sems[j % 2])
        if j > 0:
            copies[j - 1].wait()
            pltpu.sync_copy(bufs[(j - 1) % 2],
                            out_hbm.at[pl.ds(wid * _PER_W + (j - 1) * 128, 128)])
    copies[_CHUNKS - 1].wait()
    pltpu.sync_copy(bufs[(_CHUNKS - 1) % 2],
                    out_hbm.at[pl.ds(wid * _PER_W + (_CHUNKS - 1) * 128, 128)])


@functools.cache
def _sc_gather():
    return pl.kernel(
        _sc_gather_body,
        mesh=plsc.VectorSubcoreMesh(core_axis_name="c", subcore_axis_name="s"),
        out_type=jax.ShapeDtypeStruct((_NG, 32), jnp.float32),
        scratch_types=[
            pltpu.VMEM((_CHUNKS, 128), jnp.int32),
            pltpu.VMEM((128, 32), jnp.float32),
            pltpu.VMEM((128, 32), jnp.float32),
            pltpu.SemaphoreType.DMA,
            pltpu.SemaphoreType.DMA,
        ],
        compiler_params=pltpu.CompilerParams(use_tc_tiling_on_sc=False),
    )


def _gather_rows(table, idx2d):
    return _sc_gather()(table, idx2d)


def _loss_body(par_ref, gath_ref, out_ref):
    i = pl.program_id(0)

    @pl.when(i == 0)
    def _init():
        out_ref[...] = jnp.zeros((1, 1), jnp.float32)

    par = par_ref[...]
    mu = (par[:, 0:8], par[:, 8:16], par[:, 16:24])
    ls = (par[:, 24:32], par[:, 32:40], par[:, 40:48])
    mix = par[:, 48:56]
    occ = par[:, 56:64]
    mui = par[:, 64:72]
    rp = (par[:, 72:73], par[:, 73:74], par[:, 74:75])

    inv_s2 = tuple(1.0 / (jnp.exp(2.0 * l) + 1e-12) for l in ls)
    mmax = jnp.max(mix, axis=1, keepdims=True)
    lpi = mix - mmax - jnp.log(jnp.sum(jnp.exp(mix - mmax), axis=1,
                                       keepdims=True))

    gath = gath_ref[...]
    mdn_part = jnp.float32(0.0)
    int_part = jnp.float32(0.0)
    for t in range(_T):
        b = 32 * t
        g3 = gath[:, b + 3:b + 4]
        g7 = gath[:, b + 7:b + 8]
        g11 = gath[:, b + 11:b + 12]
        g15 = gath[:, b + 15:b + 16]
        gt_int = (g3 + g7 + g11 + g15) * 0.25
        y = (gath[:, b + 18:b + 19] - rp[2],
             gath[:, b + 17:b + 18] - rp[1],
             gath[:, b + 16:b + 17] - rp[0])
        quad = sum(((y[d] - mu[d]) ** 2) * inv_s2[d] + 2.0 * ls[d]
                   for d in range(3))
        logn = -0.5 * (quad + 3.0 * _LOG2PI)
        lmix = logn + lpi
        mx = jnp.max(lmix, axis=1, keepdims=True)
        e = jnp.exp(lmix - mx)
        se = jnp.sum(e, axis=1, keepdims=True)
        mdn_part += jnp.sum(mx + jnp.log(se))
        int_part += jnp.sum((e / se) * jnp.abs(mui - gt_int))

    occ_any = jnp.max(occ, axis=1)
    z = -occ_any
    occ_part = jnp.sum(jnp.maximum(z, 0.0) + jnp.log(1.0 + jnp.exp(-jnp.abs(z))))

    total = ((_W_OCC / _NR) * occ_part
             + (-_W_MDN / (_NR * _T)) * mdn_part
             + (_W_INT / (_NR * _T * _K)) * int_part)
    out_ref[...] += jnp.reshape(total, (1, 1))


def _loss(par, gath):
    return pl.pallas_call(
        _loss_body,
        grid=(_NR // _RL,),
        in_specs=[
            pl.BlockSpec((_RL, 80), lambda i: (i, 0)),
            pl.BlockSpec((_RL, 32 * _T), lambda i: (i, 0)),
        ],
        out_specs=pl.BlockSpec((1, 1), lambda i: (0, 0)),
        out_shape=jax.ShapeDtypeStruct((1, 1), jnp.float32),
    )(par, gath)


def kernel(mu_off, log_sig_off, mu_int, occ_logit, mix_logit,
           radar_indices, radar_features, lidar_indices, lidar_features):
    rpos = radar_indices[:, 1:].astype(jnp.float32)
    lpos = lidar_indices[:, 1:].astype(jnp.float32)
    padc = _NLP - _NL

    raug = jnp.concatenate(
        [-2.0 * rpos, jnp.zeros((_NR, 5), jnp.float32)], axis=1)
    raug = jnp.pad(raug, ((0, _NRP - _NR), (0, 0)))

    lmat = jnp.concatenate([
        jnp.pad(lpos.T, ((0, 0), (0, padc))),
        jnp.zeros((1, _NLP), jnp.float32),
        jnp.pad(jnp.zeros((1, _NL), jnp.float32), ((0, 0), (0, padc)),
                constant_values=_PAD_S),
        jnp.zeros((3, _NLP), jnp.float32),
    ], axis=0)

    nn_idx = _select(raug, lmat)

    table = jnp.concatenate([
        lidar_features, lpos, jnp.zeros((_NL, 13), jnp.float32),
    ], axis=1)
    idx_flat = nn_idx[:_NR, :_T].reshape(_NR * _T)
    idx2d = jnp.pad(idx_flat, (0, _NG - _NR * _T)).reshape(_NG // 128, 128)

    gath = _gather_rows(table, idx2d)
    gath = gath[:_NR * _T].reshape(_NR, 32 * _T)

    mu_t = mu_off.transpose(0, 2, 1).reshape(_NR, 24)
    ls_t = log_sig_off.transpose(0, 2, 1).reshape(_NR, 24)
    par = jnp.concatenate([
        mu_t, ls_t,
        mix_logit[..., 0], occ_logit[..., 0], mu_int[..., 0],
        rpos, jnp.zeros((_NR, 5), jnp.float32),
    ], axis=1)

    out = _loss(par, gath)
    return out[0, 0]


# speedup vs baseline: 1.4495x; 1.0173x over previous
import functools
import numpy as np
import jax
from jax import lax
import jax.numpy as jnp
from jax.experimental import pallas as pl
from jax.experimental.pallas import tpu as pltpu
from jax.experimental.pallas import tpu_sc as plsc

_NR = 5000
_NRP = 5120
_NL = 20000
_NLP = 20480
_NLH = _NLP // 2
_K = 8
_T = 10
_RS = 128
_RL = 200
_LOG2PI = float(np.log(2.0 * np.pi))
_W_OCC = 0.2
_W_MDN = 1.0
_W_INT = 0.1
_PAD_S = 65535.0

_NG = 65536
_PER_W = _NG // 32
_CHUNKS = _PER_W // 128


def _select_body(raug_ref, lmat_ref, idx_ref):
    lm = lmat_ref[...]
    l0 = lm[0:1, :]
    l1 = lm[1:2, :]
    l2c = lm[2:3, :]
    lpad = lm[4:5, :]
    lsq = l0 * l0 + l1 * l1 + l2c * l2c + lpad

    rdl = jnp.dot(raug_ref[...], lm, preferred_element_type=jnp.float32)
    s = lsq + rdl
    packed = s.astype(jnp.int32) * 32768 + jax.lax.broadcasted_iota(
        jnp.int32, s.shape, 1)

    a = packed[:, :_NLH]
    b = packed[:, _NLH:]
    rmin = jnp.minimum(a, b)
    pmax = jnp.maximum(a, b)
    imax = jnp.int32(2147483647)
    for t in range(_T):
        m = jnp.min(rmin, axis=1, keepdims=True)
        idx_ref[:, t:t + 1] = m & 32767
        pos = rmin == m
        rmin = jnp.where(pos, pmax, rmin)
        pmax = jnp.where(pos, imax, pmax)


def _select(raug, lmat):
    return pl.pallas_call(
        _select_body,
        grid=(_NRP // _RS,),
        in_specs=[
            pl.BlockSpec((_RS, 8), lambda i: (i, 0)),
            pl.BlockSpec((8, _NLP), lambda i: (0, 0)),
        ],
        out_specs=pl.BlockSpec((_RS, 16), lambda i: (i, 0)),
        out_shape=jax.ShapeDtypeStruct((_NRP, 16), jnp.int32),
    )(raug, lmat)


def _sc_gather_body(table_hbm, idx_hbm, out_hbm, idx_v, rows_a, rows_b,
                    sem_a, sem_b):
    wid = lax.axis_index("s") * 2 + lax.axis_index("c")
    pltpu.sync_copy(idx_hbm.at[pl.ds(wid * _CHUNKS, _CHUNKS)], idx_v)
    bufs = (rows_a, rows_b)
    sems = (sem_a, sem_b)
    copies = [None] * _CHUNKS
    for j in range(_CHUNKS):
        copies[j] = pltpu.async_copy(table_hbm.at[idx_v.at[j]],
                                     bufs[j % 2], sems[j % 2])
        if j > 0:
            copies[j - 1].wait()
            pltpu.sync_copy(bufs[(j - 1) % 2],
                            out_hbm.at[pl.ds(wid * _PER_W + (j - 1) * 128, 128)])
    copies[_CHUNKS - 1].wait()
    pltpu.sync_copy(bufs[(_CHUNKS - 1) % 2],
                    out_hbm.at[pl.ds(wid * _PER_W + (_CHUNKS - 1) * 128, 128)])


@functools.cache
def _sc_gather():
    return pl.kernel(
        _sc_gather_body,
        mesh=plsc.VectorSubcoreMesh(core_axis_name="c", subcore_axis_name="s"),
        out_type=jax.ShapeDtypeStruct((_NG, 32), jnp.float32),
        scratch_types=[
            pltpu.VMEM((_CHUNKS, 128), jnp.int32),
            pltpu.VMEM((128, 32), jnp.float32),
            pltpu.VMEM((128, 32), jnp.float32),
            pltpu.SemaphoreType.DMA,
            pltpu.SemaphoreType.DMA,
        ],
        compiler_params=pltpu.CompilerParams(use_tc_tiling_on_sc=False),
    )


def _gather_rows(table, idx2d):
    return _sc_gather()(table, idx2d)


def _loss_body(par_ref, gath_ref, out_ref):
    i = pl.program_id(0)

    @pl.when(i == 0)
    def _init():
        out_ref[...] = jnp.zeros((1, 1), jnp.float32)

    par = par_ref[...]
    mu = (par[:, 0:8], par[:, 8:16], par[:, 16:24])
    ls = (par[:, 24:32], par[:, 32:40], par[:, 40:48])
    mix = par[:, 48:56]
    occ = par[:, 56:64]
    mui = par[:, 64:72]
    rp = (par[:, 72:73], par[:, 73:74], par[:, 74:75])

    inv_s2 = tuple(1.0 / (jnp.exp(2.0 * l) + 1e-12) for l in ls)
    mmax = jnp.max(mix, axis=1, keepdims=True)
    lpi = mix - mmax - jnp.log(jnp.sum(jnp.exp(mix - mmax), axis=1,
                                       keepdims=True))

    gath = gath_ref[...]
    mdn_part = jnp.float32(0.0)
    int_part = jnp.float32(0.0)
    for t in range(_T):
        b = 32 * t
        g3 = gath[:, b + 3:b + 4]
        g7 = gath[:, b + 7:b + 8]
        g11 = gath[:, b + 11:b + 12]
        g15 = gath[:, b + 15:b + 16]
        gt_int = (g3 + g7 + g11 + g15) * 0.25
        y = (gath[:, b + 18:b + 19] - rp[2],
             gath[:, b + 17:b + 18] - rp[1],
             gath[:, b + 16:b + 17] - rp[0])
        quad = sum(((y[d] - mu[d]) ** 2) * inv_s2[d] + 2.0 * ls[d]
                   for d in range(3))
        logn = -0.5 * (quad + 3.0 * _LOG2PI)
        lmix = logn + lpi
        mx = jnp.max(lmix, axis=1, keepdims=True)
        e = jnp.exp(lmix - mx)
        se = jnp.sum(e, axis=1, keepdims=True)
        mdn_part += jnp.sum(mx + jnp.log(se))
        int_part += jnp.sum((e / se) * jnp.abs(mui - gt_int))

    occ_any = jnp.max(occ, axis=1)
    z = -occ_any
    occ_part = jnp.sum(jnp.maximum(z, 0.0) + jnp.log(1.0 + jnp.exp(-jnp.abs(z))))

    total = ((_W_OCC / _NR) * occ_part
             + (-_W_MDN / (_NR * _T)) * mdn_part
             + (_W_INT / (_NR * _T * _K)) * int_part)
    out_ref[...] += jnp.reshape(total, (1, 1))


def _loss(par, gath):
    return pl.pallas_call(
        _loss_body,
        grid=(_NR // _RL,),
        in_specs=[
            pl.BlockSpec((_RL, 80), lambda i: (i, 0)),
            pl.BlockSpec((_RL, 32 * _T), lambda i: (i, 0)),
        ],
        out_specs=pl.BlockSpec((1, 1), lambda i: (0, 0)),
        out_shape=jax.ShapeDtypeStruct((1, 1), jnp.float32),
    )(par, gath)


def kernel(mu_off, log_sig_off, mu_int, occ_logit, mix_logit,
           radar_indices, radar_features, lidar_indices, lidar_features):
    rpos = radar_indices[:, 1:].astype(jnp.float32)
    lpos = lidar_indices[:, 1:].astype(jnp.float32)
    padc = _NLP - _NL

    raug = jnp.concatenate(
        [-2.0 * rpos, jnp.zeros((_NR, 5), jnp.float32)], axis=1)
    raug = jnp.pad(raug, ((0, _NRP - _NR), (0, 0)))

    lmat = jnp.concatenate([
        jnp.pad(lpos.T, ((0, 0), (0, padc))),
        jnp.zeros((1, _NLP), jnp.float32),
        jnp.pad(jnp.zeros((1, _NL), jnp.float32), ((0, 0), (0, padc)),
                constant_values=_PAD_S),
        jnp.zeros((3, _NLP), jnp.float32),
    ], axis=0)

    nn_idx = _select(raug, lmat)

    table = jnp.concatenate([
        lidar_features, lpos, jnp.zeros((_NL, 13), jnp.float32),
    ], axis=1)
    idx_flat = nn_idx[:_NR, :_T].reshape(_NR * _T)
    idx2d = jnp.pad(idx_flat, (0, _NG - _NR * _T)).reshape(_NG // 128, 128)

    gath = _gather_rows(table, idx2d)
    gath = gath[:_NR * _T].reshape(_NR, 32 * _T)

    mu_t = mu_off.transpose(0, 2, 1).reshape(_NR, 24)
    ls_t = log_sig_off.transpose(0, 2, 1).reshape(_NR, 24)
    par = jnp.concatenate([
        mu_t, ls_t,
        mix_logit[..., 0], occ_logit[..., 0], mu_int[..., 0],
        rpos, jnp.zeros((_NR, 5), jnp.float32),
    ], axis=1)

    out = _loss(par, gath)
    return out[0, 0]


# speedup vs baseline: 1.4518x; 1.0016x over previous
import functools
import numpy as np
import jax
from jax import lax
import jax.numpy as jnp
from jax.experimental import pallas as pl
from jax.experimental.pallas import tpu as pltpu
from jax.experimental.pallas import tpu_sc as plsc

_NR = 5000
_NRP = 5120
_NL = 20000
_NLP = 20480
_NLH = _NLP // 2
_K = 8
_T = 10
_RS = 128
_RL = 200
_LOG2PI = float(np.log(2.0 * np.pi))
_W_OCC = 0.2
_W_MDN = 1.0
_W_INT = 0.1
_PAD_S = 65535.0

_NG = 65536
_PER_W = _NG // 32
_CHUNKS = _PER_W // 128


def _select_body(raug_ref, lmat_ref, idx_ref):
    lm = lmat_ref[...]
    l0 = lm[0:1, :]
    l1 = lm[1:2, :]
    l2c = lm[2:3, :]
    lpad = lm[4:5, :]
    lsq = l0 * l0 + l1 * l1 + l2c * l2c + lpad

    rdl = jnp.dot(raug_ref[...], lm, preferred_element_type=jnp.float32)
    s = lsq + rdl
    packed = s.astype(jnp.int32) * 32768 + jax.lax.broadcasted_iota(
        jnp.int32, s.shape, 1)

    a = packed[:, :_NLH]
    b = packed[:, _NLH:]
    rmin = jnp.minimum(a, b)
    pmax = jnp.maximum(a, b)
    imax = jnp.int32(2147483647)
    for t in range(_T):
        m = jnp.min(rmin, axis=1, keepdims=True)
        idx_ref[:, t:t + 1] = m & 32767
        pos = rmin == m
        rmin = jnp.where(pos, pmax, rmin)
        pmax = jnp.where(pos, imax, pmax)


def _select(raug, lmat):
    return pl.pallas_call(
        _select_body,
        grid=(_NRP // _RS,),
        in_specs=[
            pl.BlockSpec((_RS, 8), lambda i: (i, 0)),
            pl.BlockSpec((8, _NLP), lambda i: (0, 0)),
        ],
        out_specs=pl.BlockSpec((_RS, 16), lambda i: (i, 0)),
        out_shape=jax.ShapeDtypeStruct((_NRP, 16), jnp.int32),
    )(raug, lmat)


def _sc_gather_body(table_hbm, idx_hbm, out_hbm, idx_v, rows_v, sem):
    wid = lax.axis_index("s") * 2 + lax.axis_index("c")
    pltpu.sync_copy(idx_hbm.at[pl.ds(wid * _CHUNKS, _CHUNKS)], idx_v)
    copies = [
        pltpu.async_copy(table_hbm.at[idx_v.at[j]],
                         rows_v.at[pl.ds(j * 128, 128)], sem)
        for j in range(_CHUNKS)
    ]
    for c in copies:
        c.wait()
    pltpu.sync_copy(rows_v, out_hbm.at[pl.ds(wid * _PER_W, _PER_W)])


@functools.cache
def _sc_gather():
    return pl.kernel(
        _sc_gather_body,
        mesh=plsc.VectorSubcoreMesh(core_axis_name="c", subcore_axis_name="s"),
        out_type=jax.ShapeDtypeStruct((_NG, 32), jnp.float32),
        scratch_types=[
            pltpu.VMEM((_CHUNKS, 128), jnp.int32),
            pltpu.VMEM((_PER_W, 32), jnp.float32),
            pltpu.SemaphoreType.DMA,
        ],
        compiler_params=pltpu.CompilerParams(use_tc_tiling_on_sc=False),
    )


def _gather_rows(table, idx2d):
    return _sc_gather()(table, idx2d)


def _loss_body(par_ref, gath_ref, out_ref):
    i = pl.program_id(0)

    @pl.when(i == 0)
    def _init():
        out_ref[...] = jnp.zeros((1, 1), jnp.float32)

    par = par_ref[...]
    mu = (par[:, 0:8], par[:, 8:16], par[:, 16:24])
    ls = (par[:, 24:32], par[:, 32:40], par[:, 40:48])
    mix = par[:, 48:56]
    occ = par[:, 56:64]
    mui = par[:, 64:72]
    rp = (par[:, 72:73], par[:, 73:74], par[:, 74:75])

    inv_s2 = tuple(1.0 / (jnp.exp(2.0 * l) + 1e-12) for l in ls)
    mmax = jnp.max(mix, axis=1, keepdims=True)
    lpi = mix - mmax - jnp.log(jnp.sum(jnp.exp(mix - mmax), axis=1,
                                       keepdims=True))

    gath = gath_ref[...]
    mdn_part = jnp.float32(0.0)
    int_part = jnp.float32(0.0)
    for t in range(_T):
        b = 32 * t
        g3 = gath[:, b + 3:b + 4]
        g7 = gath[:, b + 7:b + 8]
        g11 = gath[:, b + 11:b + 12]
        g15 = gath[:, b + 15:b + 16]
        gt_int = (g3 + g7 + g11 + g15) * 0.25
        y = (gath[:, b + 18:b + 19] - rp[2],
             gath[:, b + 17:b + 18] - rp[1],
             gath[:, b + 16:b + 17] - rp[0])
        quad = sum(((y[d] - mu[d]) ** 2) * inv_s2[d] + 2.0 * ls[d]
                   for d in range(3))
        logn = -0.5 * (quad + 3.0 * _LOG2PI)
        lmix = logn + lpi
        mx = jnp.max(lmix, axis=1, keepdims=True)
        e = jnp.exp(lmix - mx)
        se = jnp.sum(e, axis=1, keepdims=True)
        mdn_part += jnp.sum(mx + jnp.log(se))
        int_part += jnp.sum((e / se) * jnp.abs(mui - gt_int))

    occ_any = jnp.max(occ, axis=1)
    z = -occ_any
    occ_part = jnp.sum(jnp.maximum(z, 0.0) + jnp.log(1.0 + jnp.exp(-jnp.abs(z))))

    total = ((_W_OCC / _NR) * occ_part
             + (-_W_MDN / (_NR * _T)) * mdn_part
             + (_W_INT / (_NR * _T * _K)) * int_part)
    out_ref[...] += jnp.reshape(total, (1, 1))


def _loss(par, gath):
    return pl.pallas_call(
        _loss_body,
        grid=(_NR // _RL,),
        in_specs=[
            pl.BlockSpec((_RL, 80), lambda i: (i, 0)),
            pl.BlockSpec((_RL, 32 * _T), lambda i: (i, 0)),
        ],
        out_specs=pl.BlockSpec((1, 1), lambda i: (0, 0)),
        out_shape=jax.ShapeDtypeStruct((1, 1), jnp.float32),
    )(par, gath)


def kernel(mu_off, log_sig_off, mu_int, occ_logit, mix_logit,
           radar_indices, radar_features, lidar_indices, lidar_features):
    rpos = radar_indices[:, 1:].astype(jnp.float32)
    lpos = lidar_indices[:, 1:].astype(jnp.float32)
    padc = _NLP - _NL

    raug = jnp.concatenate(
        [-2.0 * rpos, jnp.zeros((_NR, 5), jnp.float32)], axis=1)
    raug = jnp.pad(raug, ((0, _NRP - _NR), (0, 0)))

    lmat = jnp.concatenate([
        jnp.pad(lpos.T, ((0, 0), (0, padc))),
        jnp.zeros((1, _NLP), jnp.float32),
        jnp.pad(jnp.zeros((1, _NL), jnp.float32), ((0, 0), (0, padc)),
                constant_values=_PAD_S),
        jnp.zeros((3, _NLP), jnp.float32),
    ], axis=0)

    nn_idx = _select(raug, lmat)

    table = jnp.concatenate([
        lidar_features, lpos, jnp.zeros((_NL, 13), jnp.float32),
    ], axis=1)
    idx_flat = nn_idx[:_NR, :_T].reshape(_NR * _T)
    idx2d = jnp.pad(idx_flat, (0, _NG - _NR * _T)).reshape(_NG // 128, 128)

    gath = _gather_rows(table, idx2d)
    gath = gath[:_NR * _T].reshape(_NR, 32 * _T)

    mu_t = mu_off.transpose(0, 2, 1).reshape(_NR, 24)
    ls_t = log_sig_off.transpose(0, 2, 1).reshape(_NR, 24)
    par = jnp.concatenate([
        mu_t, ls_t,
        mix_logit[..., 0], occ_logit[..., 0], mu_int[..., 0],
        rpos, jnp.zeros((_NR, 5), jnp.float32),
    ], axis=1)

    out = _loss(par, gath)
    return out[0, 0]


# speedup vs baseline: 1.6878x; 1.1626x over previous
import functools
import numpy as np
import jax
from jax import lax
import jax.numpy as jnp
from jax.experimental import pallas as pl
from jax.experimental.pallas import tpu as pltpu
from jax.experimental.pallas import tpu_sc as plsc

_NR = 5000
_NRP = 5120
_NL = 20000
_NLP = 20480
_NLH = _NLP // 2
_K = 8
_T = 10
_RS = 128
_RL = 512
_LOG2PI = float(np.log(2.0 * np.pi))
_W_OCC = 0.2
_W_MDN = 1.0
_W_INT = 0.1
_PAD_S = 65535.0

_NG = 65536
_PER_W = _NG // 32


def _select_body(raug_ref, lmat_ref, prow_ref, idx_ref):
    lm = lmat_ref[...]
    rdl = jnp.dot(raug_ref[...], lm, preferred_element_type=jnp.float32)
    packed = prow_ref[0:1, :] + (rdl.astype(jnp.int32) << 15)

    a = packed[:, :_NLH]
    b = packed[:, _NLH:]
    rmin = jnp.minimum(a, b)
    pmax = jnp.maximum(a, b)
    imax = jnp.int32(2147483647)
    for t in range(_T):
        m = jnp.min(rmin, axis=1, keepdims=True)
        idx_ref[:, t:t + 1] = m & 32767
        pos = rmin == m
        rmin = jnp.where(pos, pmax, rmin)
        pmax = jnp.where(pos, imax, pmax)


def _select(raug, lmat, prow):
    return pl.pallas_call(
        _select_body,
        grid=(_NRP // _RS,),
        in_specs=[
            pl.BlockSpec((_RS, 8), lambda i: (i, 0)),
            pl.BlockSpec((8, _NLP), lambda i: (0, 0)),
            pl.BlockSpec((8, _NLP), lambda i: (0, 0)),
        ],
        out_specs=pl.BlockSpec((_RS, 16), lambda i: (i, 0)),
        out_shape=jax.ShapeDtypeStruct((_NRP, 16), jnp.int32),
    )(raug, lmat, prow)


def _sc_gather_body(table_hbm, idx_hbm, out_hbm, idx_v, rows_v, sem):
    wid = lax.axis_index("s") * 2 + lax.axis_index("c")
    pltpu.sync_copy(idx_hbm.at[pl.ds(wid * _PER_W, _PER_W)], idx_v)
    pltpu.async_copy(table_hbm.at[idx_v], rows_v, sem).wait()
    pltpu.sync_copy(rows_v, out_hbm.at[pl.ds(wid * _PER_W, _PER_W)])


@functools.cache
def _sc_gather():
    return pl.kernel(
        _sc_gather_body,
        mesh=plsc.VectorSubcoreMesh(core_axis_name="c", subcore_axis_name="s"),
        out_type=jax.ShapeDtypeStruct((_NG, 32), jnp.float32),
        scratch_types=[
            pltpu.VMEM((_PER_W,), jnp.int32),
            pltpu.VMEM((_PER_W, 32), jnp.float32),
            pltpu.SemaphoreType.DMA,
        ],
        compiler_params=pltpu.CompilerParams(use_tc_tiling_on_sc=False),
    )


def _gather_rows(table, idx1d):
    return _sc_gather()(table, idx1d)


def _loss_body(part_ref, gatht_ref, out_ref):
    i = pl.program_id(0)

    @pl.when(i == 0)
    def _init():
        out_ref[...] = jnp.zeros((1, 1), jnp.float32)

    par = part_ref[...]
    mu = (par[0:8, :], par[8:16, :], par[16:24, :])
    ls = (par[24:32, :], par[32:40, :], par[40:48, :])
    mix = par[48:56, :]
    occ = par[56:64, :]
    mui = par[64:72, :]
    rp = (par[72:73, :], par[73:74, :], par[74:75, :])

    col = jax.lax.broadcasted_iota(jnp.int32, (1, _RL), 1) + i * _RL
    rowm = (col < _NR).astype(jnp.float32)

    inv_s2 = tuple(1.0 / (jnp.exp(2.0 * l) + 1e-12) for l in ls)
    mmax = jnp.max(mix, axis=0, keepdims=True)
    lpi = mix - mmax - jnp.log(jnp.sum(jnp.exp(mix - mmax), axis=0,
                                       keepdims=True))

    g = gatht_ref[...]
    mdn_vec = jnp.zeros((1, _RL), jnp.float32)
    int_vec = jnp.zeros((1, _RL), jnp.float32)
    for t in range(_T):
        b = 32 * t
        gt_int = (g[b + 3:b + 4, :] + g[b + 7:b + 8, :]
                  + g[b + 11:b + 12, :] + g[b + 15:b + 16, :]) * 0.25
        y = (g[b + 18:b + 19, :] - rp[2],
             g[b + 17:b + 18, :] - rp[1],
             g[b + 16:b + 17, :] - rp[0])
        quad = sum(((y[d] - mu[d]) ** 2) * inv_s2[d] + 2.0 * ls[d]
                   for d in range(3))
        lmix = -0.5 * (quad + 3.0 * _LOG2PI) + lpi
        mx = jnp.max(lmix, axis=0, keepdims=True)
        e = jnp.exp(lmix - mx)
        se = jnp.sum(e, axis=0, keepdims=True)
        mdn_vec += mx + jnp.log(se)
        int_vec += jnp.sum(e * jnp.abs(mui - gt_int), axis=0, keepdims=True) / se

    occ_any = jnp.max(occ, axis=0, keepdims=True)
    z = -occ_any
    occ_vec = jnp.maximum(z, 0.0) + jnp.log(1.0 + jnp.exp(-jnp.abs(z)))

    total = ((_W_OCC / _NR) * jnp.sum(occ_vec * rowm)
             + (-_W_MDN / (_NR * _T)) * jnp.sum(mdn_vec * rowm)
             + (_W_INT / (_NR * _T * _K)) * jnp.sum(int_vec * rowm))
    out_ref[...] += jnp.reshape(total, (1, 1))


def _loss(part, gatht):
    return pl.pallas_call(
        _loss_body,
        grid=(_NRP // _RL,),
        in_specs=[
            pl.BlockSpec((80, _RL), lambda i: (0, i)),
            pl.BlockSpec((32 * _T, _RL), lambda i: (0, i)),
        ],
        out_specs=pl.BlockSpec((1, 1), lambda i: (0, 0)),
        out_shape=jax.ShapeDtypeStruct((1, 1), jnp.float32),
    )(part, gatht)


def kernel(mu_off, log_sig_off, mu_int, occ_logit, mix_logit,
           radar_indices, radar_features, lidar_indices, lidar_features):
    rpos = radar_indices[:, 1:].astype(jnp.float32)
    lpos = lidar_indices[:, 1:].astype(jnp.float32)
    padc = _NLP - _NL

    raug = jnp.concatenate(
        [-2.0 * rpos, jnp.zeros((_NR, 5), jnp.float32)], axis=1)
    raug = jnp.pad(raug, ((0, _NRP - _NR), (0, 0)))

    lmat = jnp.concatenate([
        jnp.pad(lpos.T, ((0, 0), (0, padc))),
        jnp.zeros((5, _NLP), jnp.float32),
    ], axis=0)

    lsq = (lpos * lpos).sum(axis=1)
    lsq = jnp.pad(lsq, (0, padc), constant_values=_PAD_S)
    prow = (lsq.astype(jnp.int32) << 15) + jnp.arange(_NLP, dtype=jnp.int32)
    prow = jnp.broadcast_to(prow[None, :], (8, _NLP))

    nn_idx = _select(raug, lmat, prow)

    table = jnp.concatenate([
        lidar_features, lpos, jnp.zeros((_NL, 13), jnp.float32),
    ], axis=1)
    idx_flat = nn_idx[:_NR, :_T].reshape(_NR * _T)
    idx1d = jnp.pad(idx_flat, (0, _NG - _NR * _T))

    gath = _gather_rows(table, idx1d)
    gath = gath[:_NR * _T].reshape(_NR, 32 * _T)
    gatht = jnp.pad(gath, ((0, _NRP - _NR), (0, 0))).T

    mu_t = mu_off.transpose(0, 2, 1).reshape(_NR, 24)
    ls_t = log_sig_off.transpose(0, 2, 1).reshape(_NR, 24)
    par = jnp.concatenate([
        mu_t, ls_t,
        mix_logit[..., 0], occ_logit[..., 0], mu_int[..., 0],
        rpos, jnp.zeros((_NR, 5), jnp.float32),
    ], axis=1)
    part = jnp.pad(par, ((0, _NRP - _NR), (0, 0))).T

    out = _loss(part, gatht)
    return out[0, 0]


# speedup vs baseline: 1.6897x; 1.0011x over previous
import functools
import numpy as np
import jax
from jax import lax
import jax.numpy as jnp
from jax.experimental import pallas as pl
from jax.experimental.pallas import tpu as pltpu
from jax.experimental.pallas import tpu_sc as plsc

_NR = 5000
_NRP = 5120
_NL = 20000
_NLP = 20480
_NLH = _NLP // 2
_K = 8
_T = 10
_RS = 128
_RL = 512
_LOG2PI = float(np.log(2.0 * np.pi))
_W_OCC = 0.2
_W_MDN = 1.0
_W_INT = 0.1
_PAD_S = 65535.0

_NG = 65536
_PER_W = _NG // 32


def _select_body(raug_ref, lmat_ref, prow_ref, idx_ref):
    lm = lmat_ref[...]
    rdl = jnp.dot(raug_ref[...], lm, preferred_element_type=jnp.float32)
    packed = prow_ref[...] + (rdl.astype(jnp.int32) << 15)

    a = packed[:, :_NLH]
    b = packed[:, _NLH:]
    rmin = jnp.minimum(a, b)
    pmax = jnp.maximum(a, b)
    imax = jnp.int32(2147483647)
    for t in range(_T):
        m = jnp.min(rmin, axis=1, keepdims=True)
        idx_ref[:, t:t + 1] = m & 32767
        pos = rmin == m
        rmin = jnp.where(pos, pmax, rmin)
        pmax = jnp.where(pos, imax, pmax)


def _select(raug, lmat, prow):
    return pl.pallas_call(
        _select_body,
        grid=(_NRP // _RS,),
        in_specs=[
            pl.BlockSpec((_RS, 8), lambda i: (i, 0)),
            pl.BlockSpec((8, _NLP), lambda i: (0, 0)),
            pl.BlockSpec((1, _NLP), lambda i: (0, 0)),
        ],
        out_specs=pl.BlockSpec((_RS, 16), lambda i: (i, 0)),
        out_shape=jax.ShapeDtypeStruct((_NRP, 16), jnp.int32),
    )(raug, lmat, prow)


def _sc_gather_body(table_hbm, idx_hbm, out_hbm, idx_v, rows_v, sem):
    wid = lax.axis_index("s") * 2 + lax.axis_index("c")
    pltpu.sync_copy(idx_hbm.at[pl.ds(wid * _PER_W, _PER_W)], idx_v)
    pltpu.async_copy(table_hbm.at[idx_v], rows_v, sem).wait()
    pltpu.sync_copy(rows_v, out_hbm.at[pl.ds(wid * _PER_W, _PER_W)])


@functools.cache
def _sc_gather():
    return pl.kernel(
        _sc_gather_body,
        mesh=plsc.VectorSubcoreMesh(core_axis_name="c", subcore_axis_name="s"),
        out_type=jax.ShapeDtypeStruct((_NG, 32), jnp.float32),
        scratch_types=[
            pltpu.VMEM((_PER_W,), jnp.int32),
            pltpu.VMEM((_PER_W, 32), jnp.float32),
            pltpu.SemaphoreType.DMA,
        ],
        compiler_params=pltpu.CompilerParams(use_tc_tiling_on_sc=False),
    )


def _gather_rows(table, idx1d):
    return _sc_gather()(table, idx1d)


def _loss_body(part_ref, gatht_ref, out_ref):
    i = pl.program_id(0)

    @pl.when(i == 0)
    def _init():
        out_ref[...] = jnp.zeros((1, 1), jnp.float32)

    par = part_ref[...]
    mu = (par[0:8, :], par[8:16, :], par[16:24, :])
    ls = (par[24:32, :], par[32:40, :], par[40:48, :])
    mix = par[48:56, :]
    occ = par[56:64, :]
    mui = par[64:72, :]
    rp = (par[72:73, :], par[73:74, :], par[74:75, :])

    col = jax.lax.broadcasted_iota(jnp.int32, (1, _RL), 1) + i * _RL
    rowm = (col < _NR).astype(jnp.float32)

    inv_s2 = tuple(1.0 / (jnp.exp(2.0 * l) + 1e-12) for l in ls)
    mmax = jnp.max(mix, axis=0, keepdims=True)
    lpi = mix - mmax - jnp.log(jnp.sum(jnp.exp(mix - mmax), axis=0,
                                       keepdims=True))

    g = gatht_ref[...]
    mdn_vec = jnp.zeros((1, _RL), jnp.float32)
    int_vec = jnp.zeros((1, _RL), jnp.float32)
    for t in range(_T):
        b = 32 * t
        gt_int = (g[b + 3:b + 4, :] + g[b + 7:b + 8, :]
                  + g[b + 11:b + 12, :] + g[b + 15:b + 16, :]) * 0.25
        y = (g[b + 18:b + 19, :] - rp[2],
             g[b + 17:b + 18, :] - rp[1],
             g[b + 16:b + 17, :] - rp[0])
        quad = sum(((y[d] - mu[d]) ** 2) * inv_s2[d] + 2.0 * ls[d]
                   for d in range(3))
        lmix = -0.5 * (quad + 3.0 * _LOG2PI) + lpi
        mx = jnp.max(lmix, axis=0, keepdims=True)
        e = jnp.exp(lmix - mx)
        se = jnp.sum(e, axis=0, keepdims=True)
        mdn_vec += mx + jnp.log(se)
        int_vec += jnp.sum(e * jnp.abs(mui - gt_int), axis=0, keepdims=True) / se

    occ_any = jnp.max(occ, axis=0, keepdims=True)
    z = -occ_any
    occ_vec = jnp.maximum(z, 0.0) + jnp.log(1.0 + jnp.exp(-jnp.abs(z)))

    total = ((_W_OCC / _NR) * jnp.sum(occ_vec * rowm)
             + (-_W_MDN / (_NR * _T)) * jnp.sum(mdn_vec * rowm)
             + (_W_INT / (_NR * _T * _K)) * jnp.sum(int_vec * rowm))
    out_ref[...] += jnp.reshape(total, (1, 1))


def _loss(part, gatht):
    return pl.pallas_call(
        _loss_body,
        grid=(_NRP // _RL,),
        in_specs=[
            pl.BlockSpec((80, _RL), lambda i: (0, i)),
            pl.BlockSpec((32 * _T, _RL), lambda i: (0, i)),
        ],
        out_specs=pl.BlockSpec((1, 1), lambda i: (0, 0)),
        out_shape=jax.ShapeDtypeStruct((1, 1), jnp.float32),
    )(part, gatht)


def kernel(mu_off, log_sig_off, mu_int, occ_logit, mix_logit,
           radar_indices, radar_features, lidar_indices, lidar_features):
    rpos = radar_indices[:, 1:].astype(jnp.float32)
    lpos = lidar_indices[:, 1:].astype(jnp.float32)
    padc = _NLP - _NL

    raug = jnp.concatenate(
        [-2.0 * rpos, jnp.zeros((_NR, 5), jnp.float32)], axis=1)
    raug = jnp.pad(raug, ((0, _NRP - _NR), (0, 0)))

    lmat = jnp.concatenate([
        jnp.pad(lpos.T, ((0, 0), (0, padc))),
        jnp.zeros((5, _NLP), jnp.float32),
    ], axis=0)

    lsq = (lpos * lpos).sum(axis=1)
    lsq = jnp.pad(lsq, (0, padc), constant_values=_PAD_S)
    prow = (lsq.astype(jnp.int32) << 15) + jnp.arange(_NLP, dtype=jnp.int32)
    prow = prow[None, :]

    nn_idx = _select(raug, lmat, prow)

    table = jnp.concatenate([
        lidar_features, lpos, jnp.zeros((_NL, 13), jnp.float32),
    ], axis=1)
    idx_flat = nn_idx[:_NR, :_T].reshape(_NR * _T)
    idx1d = jnp.pad(idx_flat, (0, _NG - _NR * _T))

    gath = _gather_rows(table, idx1d)
    gath = gath[:_NR * _T].reshape(_NR, 32 * _T)
    gatht = jnp.pad(gath, ((0, _NRP - _NR), (0, 0))).T

    mu_t = mu_off.transpose(0, 2, 1).reshape(_NR, 24)
    ls_t = log_sig_off.transpose(0, 2, 1).reshape(_NR, 24)
    par = jnp.concatenate([
        mu_t, ls_t,
        mix_logit[..., 0], occ_logit[..., 0], mu_int[..., 0],
        rpos, jnp.zeros((_NR, 5), jnp.float32),
    ], axis=1)
    part = jnp.pad(par, ((0, _NRP - _NR), (0, 0))).T

    out = _loss(part, gatht)
    return out[0, 0]
